# baseline (speedup 1.0000x reference)
import functools

import jax
import jax.numpy as jnp
from jax import lax
from jax.experimental import pallas as pl
from jax.experimental.pallas import tpu as pltpu
from jax.experimental.pallas import tpu_sc as plsc

N = 10000
E = 320000
D = 128
D3 = 3 * D
NC = 2
NS = 16
NW = NC * NS
CHUNK = 128
K = 80
KI = K + 8
HK = 40
E_PAD = NW * K * CHUNK
DUMMY = N
AGG_ROWS = 10240
ZROWS = 640


def _sc_body(x_hbm, src_hbm, dst_hbm, zer_hbm, out_hbm,
             src_v, dst_v, rows0, agg_sh, sem0):
    c = lax.axis_index("c")
    s = lax.axis_index("s")
    wid = s * NC + c

    pltpu.sync_copy(zer_hbm, rows0)
    for t in range(ZROWS // CHUNK):
        pltpu.sync_copy(rows0, agg_sh.at[pl.ds(s * ZROWS + t * CHUNK, CHUNK)])
    plsc.subcore_barrier()

    pltpu.sync_copy(src_hbm.at[wid], src_v)
    pltpu.sync_copy(dst_hbm.at[wid], dst_v)

    def body(j, carry):
        pltpu.async_copy(x_hbm.at[src_v.at[j]], rows0, sem0).wait()
        pltpu.sync_copy(rows0, agg_sh.at[dst_v.at[j]], add=True)
        return carry

    lax.fori_loop(0, K, body, 0, unroll=False)
    plsc.subcore_barrier()

    pltpu.sync_copy(agg_sh.at[pl.ds(s * ZROWS, ZROWS)],
                    out_hbm.at[c, pl.ds(s * ZROWS, ZROWS)])


@functools.partial(jax.jit, static_argnums=())
def _segment_sum_sc(x, src3, dst3, zer):
    return pl.kernel(
        _sc_body,
        out_type=jax.ShapeDtypeStruct((NC, AGG_ROWS, D), jnp.float32),
        mesh=plsc.VectorSubcoreMesh(core_axis_name="c", subcore_axis_name="s"),
        scratch_types=[
            pltpu.VMEM((KI, CHUNK), jnp.int32),
            pltpu.VMEM((K, CHUNK), jnp.int32),
            pltpu.VMEM((CHUNK, D), jnp.float32),
            pltpu.MemorySpace.VMEM_SHARED((AGG_ROWS, D), jnp.float32),
            pltpu.SemaphoreType.DMA,
        ],
    )(x, src3, dst3, zer)


B = 2000
NB = N // B


def _tc_body(x_ref, pa_ref, pb_ref, u_ref,
             w1_ref, w2_ref, bn_ref, wih_ref, whh_ref, bih_ref, bhh_ref,
             wg1_ref, wg2_ref, bg_ref, gwih_ref, gwhh_ref, gbih_ref, gbhh_ref,
             xo_ref, uo_ref, acc_ref):
    i = pl.program_id(0)
    xb = x_ref[...]
    agg = pa_ref[0] + pb_ref[0]
    xo = jnp.maximum(
        jnp.dot(xb, w1_ref[...], preferred_element_type=jnp.float32)
        + jnp.dot(agg, w2_ref[...], preferred_element_type=jnp.float32)
        + bn_ref[...], 0.0)
    gi = jnp.dot(xo, wih_ref[...], preferred_element_type=jnp.float32) + bih_ref[...]
    gh = jnp.dot(xb, whh_ref[...], preferred_element_type=jnp.float32) + bhh_ref[...]
    r = jax.nn.sigmoid(gi[:, :D] + gh[:, :D])
    z = jax.nn.sigmoid(gi[:, D:2 * D] + gh[:, D:2 * D])
    n = jnp.tanh(gi[:, 2 * D:] + r * gh[:, 2 * D:])
    xn = (1.0 - z) * n + z * xb
    xo_ref[...] = xn

    @pl.when(i == 0)
    def _init():
        acc_ref[...] = jnp.zeros_like(acc_ref)

    acc_ref[...] += jnp.sum(xn, axis=0, keepdims=True)

    @pl.when(i == NB - 1)
    def _global():
        mean = acc_ref[...] * (1.0 / N)
        ub = u_ref[...]
        uo = jnp.maximum(
            jnp.dot(mean, wg1_ref[...], preferred_element_type=jnp.float32)
            + jnp.dot(ub, wg2_ref[...], preferred_element_type=jnp.float32)
            + bg_ref[...], 0.0)
        gi_u = jnp.dot(uo, gwih_ref[...], preferred_element_type=jnp.float32) + gbih_ref[...]
        gh_u = jnp.dot(ub, gwhh_ref[...], preferred_element_type=jnp.float32) + gbhh_ref[...]
        ru = jax.nn.sigmoid(gi_u[:, :D] + gh_u[:, :D])
        zu = jax.nn.sigmoid(gi_u[:, D:2 * D] + gh_u[:, D:2 * D])
        nu = jnp.tanh(gi_u[:, 2 * D:] + ru * gh_u[:, 2 * D:])
        uo_ref[...] = (1.0 - zu) * nu + zu * ub


def _dense_step_tc(x, part, u, w1, w2, b_n, wih, whh, bih, bhh,
                   wg1, wg2, b_g, gwih, gwhh, gbih, gbhh):
    const = lambda shape: pl.BlockSpec(shape, lambda i: (0, 0))
    return pl.pallas_call(
        _tc_body,
        grid=(NB,),
        in_specs=[
            pl.BlockSpec((B, D), lambda i: (i, 0)),
            pl.BlockSpec((1, B, D), lambda i: (0, i, 0)),
            pl.BlockSpec((1, B, D), lambda i: (1, i, 0)),
            const((1, D)),
            const((D, D)), const((D, D)), const((1, D)),
            const((D, D3)), const((D, D3)),
            const((1, D3)), const((1, D3)),
            const((D, D)), const((D, D)), const((1, D)),
            const((D, D3)), const((D, D3)),
            const((1, D3)), const((1, D3)),
        ],
        out_specs=[
            pl.BlockSpec((B, D), lambda i: (i, 0)),
            pl.BlockSpec((1, D), lambda i: (0, 0)),
        ],
        out_shape=[
            jax.ShapeDtypeStruct((N, D), jnp.float32),
            jax.ShapeDtypeStruct((1, D), jnp.float32),
        ],
        scratch_shapes=[pltpu.VMEM((1, D), jnp.float32)],
    )(x, part, part, u, w1, w2, b_n, wih, whh, bih, bhh,
      wg1, wg2, b_g, gwih, gwhh, gbih, gbhh)


def kernel(x, edge_index, u, batch, Wn, bn, Wg, bg,
           nw_ih, nw_hh, nb_ih, nb_hh, gw_ih, gw_hh, gb_ih, gb_hh):
    src = edge_index[0].astype(jnp.int32)
    dst = edge_index[1].astype(jnp.int32)
    pad = E_PAD - E
    src3 = jnp.concatenate([src, jnp.zeros((pad,), jnp.int32)]).reshape(NW, K, CHUNK)
    src3 = jnp.concatenate([src3, jnp.zeros((NW, KI - K, CHUNK), jnp.int32)], axis=1)
    dst3 = jnp.concatenate([dst, jnp.full((pad,), DUMMY, jnp.int32)]).reshape(NW, K, CHUNK)
    zer = jnp.zeros((CHUNK, D), jnp.float32)

    w1, w2 = Wn[:D], Wn[D:]
    wg1, wg2 = Wg[:D], Wg[D:]
    wih, whh = nw_ih.T, nw_hh.T
    gwih, gwhh = gw_ih.T, gw_hh.T
    b_n = bn.reshape(1, D)
    b_g = bg.reshape(1, D)
    bih = nb_ih.reshape(1, D3)
    bhh = nb_hh.reshape(1, D3)
    gbih = gb_ih.reshape(1, D3)
    gbhh = gb_hh.reshape(1, D3)

    us = []
    for _ in range(2):
        part = _segment_sum_sc(x, src3, dst3, zer)
        x, u = _dense_step_tc(x, part, u, w1, w2, b_n, wih, whh, bih, bhh,
                              wg1, wg2, b_g, gwih, gwhh, gbih, gbhh)
        us.append(u[:, None, :])
    return jnp.concatenate(us, axis=1)


# speedup vs baseline: 1.5614x; 1.5614x over previous
import functools

import jax
import jax.numpy as jnp
from jax import lax
from jax.experimental import pallas as pl
from jax.experimental.pallas import tpu as pltpu
from jax.experimental.pallas import tpu_sc as plsc

N = 10000
E = 320000
D = 128
D3 = 3 * D
NC = 2
NS = 16
NW = NC * NS
CHUNK = 128
K = 79
KI = K
HK = 40
E_PAD = NW * K * CHUNK
DUMMY = N
AGG_ROWS = 10240
ZROWS = 640


def _sc_body(x_hbm, src_hbm, dst_hbm, zer_hbm, out_hbm,
             src_v, dst_v, rows0, agg_sh, sem0):
    c = lax.axis_index("c")
    s = lax.axis_index("s")
    wid = s * NC + c

    pltpu.sync_copy(zer_hbm, rows0)
    for t in range(ZROWS // CHUNK):
        pltpu.sync_copy(rows0, agg_sh.at[pl.ds(s * ZROWS + t * CHUNK, CHUNK)])
    plsc.subcore_barrier()

    pltpu.sync_copy(src_hbm.at[wid], src_v)
    pltpu.sync_copy(dst_hbm.at[wid], dst_v)

    def body(j, carry):
        pltpu.async_copy(x_hbm.at[src_v.at[j]], rows0, sem0).wait()
        pltpu.sync_copy(rows0, agg_sh.at[dst_v.at[j]], add=True)
        return carry

    lax.fori_loop(0, K, body, 0)
    plsc.subcore_barrier()

    pltpu.sync_copy(agg_sh.at[pl.ds(s * ZROWS, ZROWS)],
                    out_hbm.at[c, pl.ds(s * ZROWS, ZROWS)])


@functools.partial(jax.jit, static_argnums=())
def _segment_sum_sc(x, src3, dst3, zer):
    return pl.kernel(
        _sc_body,
        out_type=jax.ShapeDtypeStruct((NC, AGG_ROWS, D), jnp.float32),
        mesh=plsc.VectorSubcoreMesh(core_axis_name="c", subcore_axis_name="s"),
        scratch_types=[
            pltpu.VMEM((KI, CHUNK), jnp.int32),
            pltpu.VMEM((K, CHUNK), jnp.int32),
            pltpu.VMEM((CHUNK, D), jnp.float32),
            pltpu.MemorySpace.VMEM_SHARED((AGG_ROWS, D), jnp.float32),
            pltpu.SemaphoreType.DMA,
        ],
    )(x, src3, dst3, zer)


B = 2000
NB = N // B


def _tc_body(x_ref, pa_ref, pb_ref, u_ref,
             w1_ref, w2_ref, bn_ref, wih_ref, whh_ref, bih_ref, bhh_ref,
             wg1_ref, wg2_ref, bg_ref, gwih_ref, gwhh_ref, gbih_ref, gbhh_ref,
             xo_ref, uo_ref, acc_ref):
    i = pl.program_id(0)
    xb = x_ref[...]
    agg = pa_ref[0] + pb_ref[0]
    xo = jnp.maximum(
        jnp.dot(xb, w1_ref[...], preferred_element_type=jnp.float32)
        + jnp.dot(agg, w2_ref[...], preferred_element_type=jnp.float32)
        + bn_ref[...], 0.0)
    gi = jnp.dot(xo, wih_ref[...], preferred_element_type=jnp.float32) + bih_ref[...]
    gh = jnp.dot(xb, whh_ref[...], preferred_element_type=jnp.float32) + bhh_ref[...]
    r = jax.nn.sigmoid(gi[:, :D] + gh[:, :D])
    z = jax.nn.sigmoid(gi[:, D:2 * D] + gh[:, D:2 * D])
    n = jnp.tanh(gi[:, 2 * D:] + r * gh[:, 2 * D:])
    xn = (1.0 - z) * n + z * xb
    xo_ref[...] = xn

    @pl.when(i == 0)
    def _init():
        acc_ref[...] = jnp.zeros_like(acc_ref)

    acc_ref[...] += jnp.sum(xn, axis=0, keepdims=True)

    @pl.when(i == NB - 1)
    def _global():
        mean = acc_ref[...] * (1.0 / N)
        ub = u_ref[...]
        uo = jnp.maximum(
            jnp.dot(mean, wg1_ref[...], preferred_element_type=jnp.float32)
            + jnp.dot(ub, wg2_ref[...], preferred_element_type=jnp.float32)
            + bg_ref[...], 0.0)
        gi_u = jnp.dot(uo, gwih_ref[...], preferred_element_type=jnp.float32) + gbih_ref[...]
        gh_u = jnp.dot(ub, gwhh_ref[...], preferred_element_type=jnp.float32) + gbhh_ref[...]
        ru = jax.nn.sigmoid(gi_u[:, :D] + gh_u[:, :D])
        zu = jax.nn.sigmoid(gi_u[:, D:2 * D] + gh_u[:, D:2 * D])
        nu = jnp.tanh(gi_u[:, 2 * D:] + ru * gh_u[:, 2 * D:])
        uo_ref[...] = (1.0 - zu) * nu + zu * ub


def _dense_step_tc(x, part, u, w1, w2, b_n, wih, whh, bih, bhh,
                   wg1, wg2, b_g, gwih, gwhh, gbih, gbhh):
    const = lambda shape: pl.BlockSpec(shape, lambda i: (0, 0))
    return pl.pallas_call(
        _tc_body,
        grid=(NB,),
        in_specs=[
            pl.BlockSpec((B, D), lambda i: (i, 0)),
            pl.BlockSpec((1, B, D), lambda i: (0, i, 0)),
            pl.BlockSpec((1, B, D), lambda i: (1, i, 0)),
            const((1, D)),
            const((D, D)), const((D, D)), const((1, D)),
            const((D, D3)), const((D, D3)),
            const((1, D3)), const((1, D3)),
            const((D, D)), const((D, D)), const((1, D)),
            const((D, D3)), const((D, D3)),
            const((1, D3)), const((1, D3)),
        ],
        out_specs=[
            pl.BlockSpec((B, D), lambda i: (i, 0)),
            pl.BlockSpec((1, D), lambda i: (0, 0)),
        ],
        out_shape=[
            jax.ShapeDtypeStruct((N, D), jnp.float32),
            jax.ShapeDtypeStruct((1, D), jnp.float32),
        ],
        scratch_shapes=[pltpu.VMEM((1, D), jnp.float32)],
    )(x, part, part, u, w1, w2, b_n, wih, whh, bih, bhh,
      wg1, wg2, b_g, gwih, gwhh, gbih, gbhh)


def kernel(x, edge_index, u, batch, Wn, bn, Wg, bg,
           nw_ih, nw_hh, nb_ih, nb_hh, gw_ih, gw_hh, gb_ih, gb_hh):
    src = edge_index[0].astype(jnp.int32)
    dst = edge_index[1].astype(jnp.int32)
    pad = E_PAD - E
    src3 = jnp.concatenate([src, jnp.zeros((pad,), jnp.int32)]).reshape(NW, K, CHUNK)
    dst3 = jnp.concatenate([dst, jnp.full((pad,), DUMMY, jnp.int32)]).reshape(NW, K, CHUNK)
    zer = jnp.zeros((CHUNK, D), jnp.float32)

    w1, w2 = Wn[:D], Wn[D:]
    wg1, wg2 = Wg[:D], Wg[D:]
    wih, whh = nw_ih.T, nw_hh.T
    gwih, gwhh = gw_ih.T, gw_hh.T
    b_n = bn.reshape(1, D)
    b_g = bg.reshape(1, D)
    bih = nb_ih.reshape(1, D3)
    bhh = nb_hh.reshape(1, D3)
    gbih = gb_ih.reshape(1, D3)
    gbhh = gb_hh.reshape(1, D3)

    us = []
    for _ in range(2):
        part = _segment_sum_sc(x, src3, dst3, zer)
        x, u = _dense_step_tc(x, part, u, w1, w2, b_n, wih, whh, bih, bhh,
                              wg1, wg2, b_g, gwih, gwhh, gbih, gbhh)
        us.append(u[:, None, :])
    return jnp.concatenate(us, axis=1)
